# baseline (speedup 1.0000x reference)
import functools

import jax
import jax.numpy as jnp
from jax import lax
from jax.experimental import pallas as pl
from jax.experimental.pallas import tpu as pltpu
from jax.experimental.pallas import tpu_sc as plsc

D_MODEL = 1024
N_TOKENS = 4 * 4096
NUM_WORKERS = 32
PER_WORKER = N_TOKENS // NUM_WORKERS
CHUNK = 32
NUM_CHUNKS = PER_WORKER // CHUNK
NBUF = 3

_mesh = plsc.VectorSubcoreMesh(core_axis_name="c", subcore_axis_name="s")


@functools.partial(
    pl.kernel,
    mesh=_mesh,
    out_type=jax.ShapeDtypeStruct((N_TOKENS, D_MODEL), jnp.float32),
    scratch_types=[
        pltpu.VMEM((PER_WORKER,), jnp.int32),
        *[pltpu.VMEM((CHUNK, D_MODEL), jnp.float32) for _ in range(NBUF)],
        *[pltpu.SemaphoreType.DMA for _ in range(2 * NBUF)],
    ],
)
def _gather_rows(table_hbm, ids_hbm, out_hbm, idx_v, *scratch):
    bufs = scratch[:NBUF]
    gsems = scratch[NBUF:2 * NBUF]
    osems = scratch[2 * NBUF:]
    wid = lax.axis_index("s") * 2 + lax.axis_index("c")
    base = wid * PER_WORKER
    pltpu.sync_copy(ids_hbm.at[pl.ds(base, PER_WORKER)], idx_v)

    def gather(j):
        idx_slice = idx_v.at[pl.ds(j * CHUNK, CHUNK)]
        return pltpu.async_copy(table_hbm.at[idx_slice], bufs[j % NBUF],
                                gsems[j % NBUF])

    def put(j):
        return pltpu.async_copy(bufs[j % NBUF],
                                out_hbm.at[pl.ds(base + j * CHUNK, CHUNK)],
                                osems[j % NBUF])

    g = [None] * NUM_CHUNKS
    o = [None] * NUM_CHUNKS
    g[0] = gather(0)
    for j in range(NUM_CHUNKS):
        if j + 1 < NUM_CHUNKS:
            if j - NBUF + 1 >= 0:
                o[j - NBUF + 1].wait()
            g[j + 1] = gather(j + 1)
        g[j].wait()
        o[j] = put(j)
    for j in range(max(0, NUM_CHUNKS - NBUF), NUM_CHUNKS):
        o[j].wait()


def kernel(input_ids, input_mask, weight):
    del input_mask
    b, s = input_ids.shape
    ids_flat = input_ids.reshape(-1).astype(jnp.int32)
    out = _gather_rows(weight, ids_flat)
    return out.reshape(b, s, D_MODEL)


# speedup vs baseline: 1.0150x; 1.0150x over previous
import functools

import jax
import jax.numpy as jnp
from jax import lax
from jax.experimental import pallas as pl
from jax.experimental.pallas import tpu as pltpu
from jax.experimental.pallas import tpu_sc as plsc

D_MODEL = 1024
N_TOKENS = 4 * 4096
NUM_WORKERS = 32
PER_WORKER = N_TOKENS // NUM_WORKERS
CHUNK = 32
NUM_CHUNKS = PER_WORKER // CHUNK
NBUF = 2

_mesh = plsc.VectorSubcoreMesh(core_axis_name="c", subcore_axis_name="s")


@functools.partial(
    pl.kernel,
    mesh=_mesh,
    out_type=jax.ShapeDtypeStruct((N_TOKENS, D_MODEL), jnp.float32),
    scratch_types=[
        pltpu.VMEM((PER_WORKER,), jnp.int32),
        *[pltpu.VMEM((CHUNK, D_MODEL), jnp.float32) for _ in range(NBUF)],
        *[pltpu.SemaphoreType.DMA for _ in range(2 * NBUF)],
    ],
)
def _gather_rows(table_hbm, ids_hbm, out_hbm, idx_v, *scratch):
    bufs = scratch[:NBUF]
    gsems = scratch[NBUF:2 * NBUF]
    osems = scratch[2 * NBUF:]
    wid = lax.axis_index("s") * 2 + lax.axis_index("c")
    base = wid * PER_WORKER
    pltpu.sync_copy(ids_hbm.at[pl.ds(base, PER_WORKER)], idx_v)

    def gather(j):
        idx_slice = idx_v.at[pl.ds(j * CHUNK, CHUNK)]
        return pltpu.async_copy(table_hbm.at[idx_slice], bufs[j % NBUF],
                                gsems[j % NBUF])

    def put(j):
        return pltpu.async_copy(bufs[j % NBUF],
                                out_hbm.at[pl.ds(base + j * CHUNK, CHUNK)],
                                osems[j % NBUF])

    g = [None] * NUM_CHUNKS
    o = [None] * NUM_CHUNKS
    g[0] = gather(0)
    for j in range(NUM_CHUNKS):
        if j + 1 < NUM_CHUNKS:
            if j - NBUF + 1 >= 0:
                o[j - NBUF + 1].wait()
            g[j + 1] = gather(j + 1)
        g[j].wait()
        o[j] = put(j)
    for j in range(max(0, NUM_CHUNKS - NBUF), NUM_CHUNKS):
        o[j].wait()


def kernel(input_ids, input_mask, weight):
    del input_mask
    b, s = input_ids.shape
    ids_flat = input_ids.reshape(-1).astype(jnp.int32)
    out = _gather_rows(weight, ids_flat)
    return out.reshape(b, s, D_MODEL)
